# baseline (speedup 1.0000x reference)
import functools

import jax
import jax.numpy as jnp
from jax import lax
from jax.experimental import pallas as pl
from jax.experimental.pallas import tpu as pltpu
from jax.experimental.pallas import tpu_sc as plsc

_B = 16384
_N_ENT = 1000000
_DIM = 64
_MARGIN = 1.0

_NC = 2
_NS = 16
_NW = _NC * _NS
_LANES = 16

_N_IDX = 4 * _B
_CW = 512
_NFULL = _N_ENT // _CW
_TAIL = _N_ENT - _NFULL * _CW
_CPW = _NFULL // _NW
_CREM = _NFULL - _CPW * _NW
_CAP = 4096
_SCN = 4096
_RCHUNK = 128
_RGRP = 16


def _scan_round(idx_hbm, idx_v, li_v, lb_v, sem, rlo, rhi, r):
    win_lo = r * _CAP

    def outer(p, tot):
        pltpu.async_copy(
            idx_hbm.at[pl.ds(p * _SCN, _SCN)], idx_v, sem).wait()

        def inner(g, tot):
            gbase = pl.multiple_of(g * _LANES, _LANES)
            lanes = lax.iota(jnp.int32, _LANES)
            vec = idx_v[pl.ds(gbase, _LANES)]
            bvec = (p * _SCN + g * _LANES) + lanes
            m = (vec >= rlo) & (vec < rhi)
            cnt = plsc.all_reduce_population_count(m)[0]
            end = tot + cnt

            @pl.when((cnt > 0) & (tot >= win_lo) & (tot < win_lo + _CAP))
            def _fast():
                off = tot - win_lo
                plsc.store_compressed(li_v.at[pl.ds(off, _LANES)], vec,
                                      mask=m)
                plsc.store_compressed(lb_v.at[pl.ds(off, _LANES)], bvec,
                                      mask=m)

            @pl.when((tot < win_lo) & (end > win_lo))
            def _straddle():
                ex = jnp.zeros((_LANES,), jnp.int32)
                for l in range(1, _LANES):
                    c_l = plsc.all_reduce_population_count(
                        m & (lanes < l))[0]
                    ex = jnp.where(lanes == l, c_l, ex)
                keep = m & ((tot + ex) >= win_lo)
                plsc.store_compressed(li_v.at[pl.ds(0, _LANES)], vec,
                                      mask=keep)
                plsc.store_compressed(lb_v.at[pl.ds(0, _LANES)], bvec,
                                      mask=keep)

            return end

        return lax.fori_loop(0, _SCN // _LANES, inner, tot)

    total = lax.fori_loop(0, _N_IDX // _SCN, outer, jnp.int32(0))
    n_r = jnp.clip(total - win_lo, 0, _CAP)
    return n_r, total


def _extract_block(blk, ncols, clo, n_r, li_v, lb_v, col_v, out_hbm, sem):
    def grp(j, carry):
        gbase = pl.multiple_of(j * _LANES, _LANES)
        ivec = li_v[pl.ds(gbase, _LANES)]
        bvec = lb_v[pl.ds(gbase, _LANES)]
        lane_ok = (gbase + lax.iota(jnp.int32, _LANES)) < n_r
        mm = (ivec >= clo) & (ivec < clo + ncols) & lane_ok
        mi = mm.astype(jnp.int32)
        any_cnt = plsc.all_reduce_population_count(mm)[0]

        @pl.when(any_cnt > 0)
        def _work():
            for l in range(_LANES):
                @pl.when(mi[l] > 0)
                def _one(l=l):
                    col = ivec[l] - clo
                    for kk in range(_DIM // _LANES):
                        g = plsc.load_gather(
                            blk,
                            [kk * _LANES + lax.iota(jnp.int32, _LANES),
                             jnp.full((_LANES,), col, jnp.int32)])
                        col_v[pl.ds(l * _DIM + kk * _LANES, _LANES)] = g
                    pltpu.make_async_copy(
                        col_v.at[pl.ds(l * _DIM, _DIM)],
                        out_hbm.at[pl.ds(bvec[l] * _DIM, _DIM)],
                        sem).start()
            for l in range(_LANES):
                @pl.when(mi[l] > 0)
                def _drain(l=l):
                    pltpu.make_async_copy(
                        col_v.at[pl.ds(l * _DIM, _DIM)],
                        out_hbm.at[pl.ds(bvec[l] * _DIM, _DIM)],
                        sem).wait()
        return carry

    lax.fori_loop(0, (n_r + _LANES - 1) // _LANES, grp, jnp.int32(0))


def _sc_gather(ent_idx, rel_idx, ent_t, rel_emb):
    mesh = plsc.VectorSubcoreMesh(core_axis_name="c", subcore_axis_name="s")

    @functools.partial(
        pl.kernel,
        out_type=(
            jax.ShapeDtypeStruct((_N_IDX * _DIM,), jnp.float32),
            jax.ShapeDtypeStruct((_B, _DIM), jnp.float32),
        ),
        mesh=mesh,
        scratch_types=[
            pltpu.VMEM((_SCN,), jnp.int32),
            pltpu.VMEM((_CAP + _LANES,), jnp.int32),
            pltpu.VMEM((_CAP + _LANES,), jnp.int32),
            pltpu.VMEM((_DIM, _CW), jnp.float32),
            pltpu.VMEM((_DIM, _TAIL), jnp.float32),
            pltpu.VMEM((_LANES * _DIM,), jnp.float32),
            pltpu.VMEM((_RCHUNK,), jnp.int32),
            pltpu.VMEM((_RCHUNK, _DIM), jnp.float32),
            pltpu.SemaphoreType.DMA,
            pltpu.SemaphoreType.DMA,
            pltpu.SemaphoreType.DMA,
            pltpu.SemaphoreType.DMA,
        ],
        compiler_params=pltpu.CompilerParams(needs_layout_passes=False),
    )
    def gather_kernel(ent_idx_hbm, rel_idx_hbm, ent_hbm, rel_hbm,
                      ent_out, rel_out,
                      idx_v, li_v, lb_v, blk0, tail_v, col_v,
                      ridx_v, rrows_v,
                      sem_scan, sem_blk, sem_col, sem_rel):
        wid = lax.axis_index("s") * _NC + lax.axis_index("c")

        rel_per_w = _B // _NW
        rbase = wid * rel_per_w

        def rel_chunk(c, carry):
            off = rbase + c * _RCHUNK
            pltpu.sync_copy(rel_idx_hbm.at[pl.ds(off, _RCHUNK)], ridx_v)

            def rel_grp(g, carry):
                gb = pl.multiple_of(g * _RGRP, _RGRP)
                vec = ridx_v[pl.ds(gb, _RGRP)]
                copies = []
                for l in range(_RGRP):
                    cp = pltpu.make_async_copy(
                        rel_hbm.at[pl.ds(vec[l], 1), :],
                        rrows_v.at[pl.ds(gb + l, 1), :],
                        sem_rel)
                    cp.start()
                    copies.append(cp)
                for cp in copies:
                    cp.wait()
                return carry

            lax.fori_loop(0, _RCHUNK // _RGRP, rel_grp, jnp.int32(0))
            pltpu.sync_copy(rrows_v, rel_out.at[pl.ds(off, _RCHUNK)])
            return carry

        lax.fori_loop(0, rel_per_w // _RCHUNK, rel_chunk, jnp.int32(0))

        nch = _CPW + jnp.where(wid < _CREM, 1, 0)
        cstart = wid * _CPW + jnp.minimum(wid, _CREM)
        rlo = cstart * _CW
        rhi = jnp.where(wid == _NW - 1, _N_ENT, (cstart + nch) * _CW)

        def stream_round(n_r):
            def chunk(k, carry):
                cp = pltpu.make_async_copy(
                    ent_hbm.at[:, pl.ds((cstart + k) * _CW, _CW)],
                    blk0, sem_blk)
                cp.start()
                cp.wait()
                clo = (cstart + k) * _CW
                _extract_block(blk0, _CW, clo, n_r, li_v, lb_v,
                               col_v, ent_out, sem_col)
                return carry

            lax.fori_loop(0, nch, chunk, jnp.int32(0))

            @pl.when(wid == _NW - 1)
            def _tail():
                cp = pltpu.make_async_copy(
                    ent_hbm.at[:, pl.ds(_NFULL * _CW, _TAIL)],
                    tail_v, sem_blk)
                cp.start()
                cp.wait()
                _extract_block(tail_v, _TAIL, _NFULL * _CW, n_r,
                               li_v, lb_v, col_v, ent_out, sem_col)

        n0, total = _scan_round(ent_idx_hbm, idx_v, li_v, lb_v,
                                sem_scan, rlo, rhi, jnp.int32(0))
        stream_round(n0)

        def extra_round(r, carry):
            n_r, _ = _scan_round(ent_idx_hbm, idx_v, li_v, lb_v,
                                 sem_scan, rlo, rhi, r)
            stream_round(n_r)
            return carry

        lax.fori_loop(1, (total + _CAP - 1) // _CAP, extra_round,
                      jnp.int32(0))

    return gather_kernel(ent_idx, rel_idx, ent_t, rel_emb)


_BLK = 2048


def _dense_body(hp_ref, tp_ref, hn_ref, tn_ref, r_ref, out_ref):
    i = pl.program_id(0)

    def nrm(x):
        n = jnp.sqrt(jnp.sum(x * x, axis=1, keepdims=True))
        return x / jnp.maximum(n, 1e-12)

    hp = nrm(hp_ref[...])
    tp = nrm(tp_ref[...])
    hn = nrm(hn_ref[...])
    tn = nrm(tn_ref[...])
    r = nrm(r_ref[...])
    pos_e = jnp.sqrt(jnp.sum((hp + r - tp) ** 2, axis=1))
    neg_e = jnp.sqrt(jnp.sum((hn + r - tn) ** 2, axis=1))
    loss = jnp.maximum(_MARGIN + pos_e - neg_e, 0.0)
    s = jnp.sum(loss)

    @pl.when(i == 0)
    def _init():
        out_ref[0, 0] = s

    @pl.when(i != 0)
    def _acc():
        out_ref[0, 0] += s

    @pl.when(i == pl.num_programs(0) - 1)
    def _final():
        out_ref[0, 0] = out_ref[0, 0] / _B


def _dense_loss(hp, tp, hn, tn, r):
    grid = _B // _BLK
    row_spec = pl.BlockSpec((_BLK, _DIM), lambda i: (i, 0))
    return pl.pallas_call(
        _dense_body,
        grid=(grid,),
        in_specs=[row_spec] * 5,
        out_specs=pl.BlockSpec((1, 1), lambda i: (0, 0),
                               memory_space=pltpu.SMEM),
        out_shape=jax.ShapeDtypeStruct((1, 1), jnp.float32),
    )(hp, tp, hn, tn, r)


def kernel(pos_pairs, neg_pairs, rels, ent_embs, alignments, rel_emb):
    ent_idx = jnp.concatenate(
        [pos_pairs[:, 0], pos_pairs[:, 1], neg_pairs[:, 0], neg_pairs[:, 1]]
    )
    rel_idx = rels[:, 0]
    ent_flat, rel_rows = _sc_gather(ent_idx, rel_idx, ent_embs.T, rel_emb)
    ent_rows = ent_flat.reshape(_N_IDX, _DIM)
    hp = ent_rows[0:_B]
    tp = ent_rows[_B:2 * _B]
    hn = ent_rows[2 * _B:3 * _B]
    tn = ent_rows[3 * _B:4 * _B]
    out = _dense_loss(hp, tp, hn, tn, rel_rows)
    return out[0, 0]


# speedup vs baseline: 1.0822x; 1.0822x over previous
import functools

import jax
import jax.numpy as jnp
from jax import lax
from jax.experimental import pallas as pl
from jax.experimental.pallas import tpu as pltpu
from jax.experimental.pallas import tpu_sc as plsc

_B = 16384
_N_ENT = 1000000
_DIM = 64
_MARGIN = 1.0

_NC = 2
_NS = 16
_NW = _NC * _NS
_LANES = 16

_N_IDX = 4 * _B
_CW = 512
_NFULL = _N_ENT // _CW
_TAIL = _N_ENT - _NFULL * _CW
_CPW = _NFULL // _NW
_CREM = _NFULL - _CPW * _NW
_CAP = 4096
_SCN = 4096
_RCHUNK = 128
_RGRP = 16


def _scan_round(idx_hbm, idx_v, li_v, lb_v, sem, rlo, rhi, r):
    win_lo = r * _CAP

    def outer(p, tot):
        pltpu.async_copy(
            idx_hbm.at[pl.ds(p * _SCN, _SCN)], idx_v, sem).wait()

        def inner(g, tot):
            gbase = pl.multiple_of(g * _LANES, _LANES)
            lanes = lax.iota(jnp.int32, _LANES)
            vec = idx_v[pl.ds(gbase, _LANES)]
            bvec = (p * _SCN + g * _LANES) + lanes
            m = (vec >= rlo) & (vec < rhi)
            cnt = plsc.all_reduce_population_count(m)[0]
            end = tot + cnt

            in_win = (tot >= win_lo) & (tot < win_lo + _CAP)
            fast = m & in_win
            off = jnp.clip(tot - win_lo, 0, _CAP)
            plsc.store_compressed(li_v.at[pl.ds(off, _LANES)], vec,
                                  mask=fast)
            plsc.store_compressed(lb_v.at[pl.ds(off, _LANES)], bvec,
                                  mask=fast)

            @pl.when((tot < win_lo) & (end > win_lo))
            def _straddle():
                ex = jnp.zeros((_LANES,), jnp.int32)
                for l in range(1, _LANES):
                    c_l = plsc.all_reduce_population_count(
                        m & (lanes < l))[0]
                    ex = jnp.where(lanes == l, c_l, ex)
                keep = m & ((tot + ex) >= win_lo)
                plsc.store_compressed(li_v.at[pl.ds(0, _LANES)], vec,
                                      mask=keep)
                plsc.store_compressed(lb_v.at[pl.ds(0, _LANES)], bvec,
                                      mask=keep)

            return end

        return lax.fori_loop(0, _SCN // _LANES, inner, tot)

    total = lax.fori_loop(0, _N_IDX // _SCN, outer, jnp.int32(0))
    n_r = jnp.clip(total - win_lo, 0, _CAP)
    return n_r, total


def _extract_block(blk, ncols, clo, n_r, li_v, lb_v, col_v, out_hbm, sem):
    def grp(j, carry):
        gbase = pl.multiple_of(j * _LANES, _LANES)
        ivec = li_v[pl.ds(gbase, _LANES)]
        lane_ok = (gbase + lax.iota(jnp.int32, _LANES)) < n_r
        mm = (ivec >= clo) & (ivec < clo + ncols) & lane_ok
        mi = mm.astype(jnp.int32)
        any_cnt = plsc.all_reduce_population_count(mm)[0]

        @pl.when(any_cnt > 0)
        def _work():
            bvec = lb_v[pl.ds(gbase, _LANES)]
            for l in range(_LANES):
                @pl.when(mi[l] > 0)
                def _one(l=l):
                    col = ivec[l] - clo
                    for kk in range(_DIM // _LANES):
                        g = plsc.load_gather(
                            blk,
                            [kk * _LANES + lax.iota(jnp.int32, _LANES),
                             jnp.full((_LANES,), col, jnp.int32)])
                        col_v[pl.ds(l * _DIM + kk * _LANES, _LANES)] = g
                    pltpu.make_async_copy(
                        col_v.at[pl.ds(l * _DIM, _DIM)],
                        out_hbm.at[pl.ds(bvec[l] * _DIM, _DIM)],
                        sem).start()
            for l in range(_LANES):
                @pl.when(mi[l] > 0)
                def _drain(l=l):
                    pltpu.make_async_copy(
                        col_v.at[pl.ds(l * _DIM, _DIM)],
                        out_hbm.at[pl.ds(bvec[l] * _DIM, _DIM)],
                        sem).wait()
        return carry

    lax.fori_loop(0, (n_r + _LANES - 1) // _LANES, grp, jnp.int32(0))


def _sc_gather(ent_idx, rel_idx, ent_t, rel_emb):
    mesh = plsc.VectorSubcoreMesh(core_axis_name="c", subcore_axis_name="s")

    @functools.partial(
        pl.kernel,
        out_type=(
            jax.ShapeDtypeStruct((_N_IDX * _DIM,), jnp.float32),
            jax.ShapeDtypeStruct((_B, _DIM), jnp.float32),
        ),
        mesh=mesh,
        scratch_types=[
            pltpu.VMEM((_SCN,), jnp.int32),
            pltpu.VMEM((_CAP + _LANES,), jnp.int32),
            pltpu.VMEM((_CAP + _LANES,), jnp.int32),
            pltpu.VMEM((_DIM, _CW), jnp.float32),
            pltpu.VMEM((_DIM, _CW), jnp.float32),
            pltpu.VMEM((_DIM, _TAIL), jnp.float32),
            pltpu.VMEM((_LANES * _DIM,), jnp.float32),
            pltpu.VMEM((_RCHUNK,), jnp.int32),
            pltpu.VMEM((_RCHUNK, _DIM), jnp.float32),
            pltpu.SemaphoreType.DMA,
            pltpu.SemaphoreType.DMA,
            pltpu.SemaphoreType.DMA,
            pltpu.SemaphoreType.DMA,
        ],
        compiler_params=pltpu.CompilerParams(needs_layout_passes=False),
    )
    def gather_kernel(ent_idx_hbm, rel_idx_hbm, ent_hbm, rel_hbm,
                      ent_out, rel_out,
                      idx_v, li_v, lb_v, blk0, blk1, tail_v, col_v,
                      ridx_v, rrows_v,
                      sem_scan, sem_blk, sem_col, sem_rel):
        wid = lax.axis_index("s") * _NC + lax.axis_index("c")

        rel_per_w = _B // _NW
        rbase = wid * rel_per_w

        def rel_chunk(c, carry):
            off = rbase + c * _RCHUNK
            pltpu.sync_copy(rel_idx_hbm.at[pl.ds(off, _RCHUNK)], ridx_v)

            def rel_grp(g, carry):
                gb = pl.multiple_of(g * _RGRP, _RGRP)
                vec = ridx_v[pl.ds(gb, _RGRP)]
                copies = []
                for l in range(_RGRP):
                    cp = pltpu.make_async_copy(
                        rel_hbm.at[pl.ds(vec[l], 1), :],
                        rrows_v.at[pl.ds(gb + l, 1), :],
                        sem_rel)
                    cp.start()
                    copies.append(cp)
                for cp in copies:
                    cp.wait()
                return carry

            lax.fori_loop(0, _RCHUNK // _RGRP, rel_grp, jnp.int32(0))
            pltpu.sync_copy(rrows_v, rel_out.at[pl.ds(off, _RCHUNK)])
            return carry

        lax.fori_loop(0, rel_per_w // _RCHUNK, rel_chunk, jnp.int32(0))

        nch = _CPW + jnp.where(wid < _CREM, 1, 0)
        cstart = wid * _CPW + jnp.minimum(wid, _CREM)
        rlo = cstart * _CW
        rhi = jnp.where(wid == _NW - 1, _N_ENT, (cstart + nch) * _CW)

        def blk_copy(k, blk):
            return pltpu.make_async_copy(
                ent_hbm.at[:, pl.ds((cstart + k) * _CW, _CW)], blk, sem_blk)

        def stream_round(n_r):
            blk_copy(0, blk0).start()

            def chunk(k, carry):
                even = k % 2 == 0
                clo = (cstart + k) * _CW

                @pl.when((k + 1 < nch) & even)
                def _p1():
                    blk_copy(k + 1, blk1).start()

                @pl.when((k + 1 < nch) & jnp.logical_not(even))
                def _p0():
                    blk_copy(k + 1, blk0).start()

                @pl.when(even)
                def _e0():
                    blk_copy(k, blk0).wait()
                    _extract_block(blk0, _CW, clo, n_r, li_v, lb_v,
                                   col_v, ent_out, sem_col)

                @pl.when(jnp.logical_not(even))
                def _e1():
                    blk_copy(k, blk1).wait()
                    _extract_block(blk1, _CW, clo, n_r, li_v, lb_v,
                                   col_v, ent_out, sem_col)
                return carry

            lax.fori_loop(0, nch, chunk, jnp.int32(0))

            @pl.when(wid == _NW - 1)
            def _tail():
                cp = pltpu.make_async_copy(
                    ent_hbm.at[:, pl.ds(_NFULL * _CW, _TAIL)],
                    tail_v, sem_blk)
                cp.start()
                cp.wait()
                _extract_block(tail_v, _TAIL, _NFULL * _CW, n_r,
                               li_v, lb_v, col_v, ent_out, sem_col)

        n0, total = _scan_round(ent_idx_hbm, idx_v, li_v, lb_v,
                                sem_scan, rlo, rhi, jnp.int32(0))
        stream_round(n0)

        def extra_round(r, carry):
            n_r, _ = _scan_round(ent_idx_hbm, idx_v, li_v, lb_v,
                                 sem_scan, rlo, rhi, r)
            stream_round(n_r)
            return carry

        lax.fori_loop(1, (total + _CAP - 1) // _CAP, extra_round,
                      jnp.int32(0))

    return gather_kernel(ent_idx, rel_idx, ent_t, rel_emb)


_BLK = 2048


def _dense_body(hp_ref, tp_ref, hn_ref, tn_ref, r_ref, out_ref):
    i = pl.program_id(0)

    def nrm(x):
        n = jnp.sqrt(jnp.sum(x * x, axis=1, keepdims=True))
        return x / jnp.maximum(n, 1e-12)

    hp = nrm(hp_ref[...])
    tp = nrm(tp_ref[...])
    hn = nrm(hn_ref[...])
    tn = nrm(tn_ref[...])
    r = nrm(r_ref[...])
    pos_e = jnp.sqrt(jnp.sum((hp + r - tp) ** 2, axis=1))
    neg_e = jnp.sqrt(jnp.sum((hn + r - tn) ** 2, axis=1))
    loss = jnp.maximum(_MARGIN + pos_e - neg_e, 0.0)
    s = jnp.sum(loss)

    @pl.when(i == 0)
    def _init():
        out_ref[0, 0] = s

    @pl.when(i != 0)
    def _acc():
        out_ref[0, 0] += s

    @pl.when(i == pl.num_programs(0) - 1)
    def _final():
        out_ref[0, 0] = out_ref[0, 0] / _B


def _dense_loss(hp, tp, hn, tn, r):
    grid = _B // _BLK
    row_spec = pl.BlockSpec((_BLK, _DIM), lambda i: (i, 0))
    return pl.pallas_call(
        _dense_body,
        grid=(grid,),
        in_specs=[row_spec] * 5,
        out_specs=pl.BlockSpec((1, 1), lambda i: (0, 0),
                               memory_space=pltpu.SMEM),
        out_shape=jax.ShapeDtypeStruct((1, 1), jnp.float32),
    )(hp, tp, hn, tn, r)


def kernel(pos_pairs, neg_pairs, rels, ent_embs, alignments, rel_emb):
    ent_idx = jnp.concatenate(
        [pos_pairs[:, 0], pos_pairs[:, 1], neg_pairs[:, 0], neg_pairs[:, 1]]
    )
    rel_idx = rels[:, 0]
    ent_flat, rel_rows = _sc_gather(ent_idx, rel_idx, ent_embs.T, rel_emb)
    ent_rows = ent_flat.reshape(_N_IDX, _DIM)
    hp = ent_rows[0:_B]
    tp = ent_rows[_B:2 * _B]
    hn = ent_rows[2 * _B:3 * _B]
    tn = ent_rows[3 * _B:4 * _B]
    out = _dense_loss(hp, tp, hn, tn, rel_rows)
    return out[0, 0]


# speedup vs baseline: 3.7690x; 3.4828x over previous
import functools

import jax
import jax.numpy as jnp
from jax import lax
from jax.experimental import pallas as pl
from jax.experimental.pallas import tpu as pltpu
from jax.experimental.pallas import tpu_sc as plsc

_B = 16384
_DIM = 64
_MARGIN = 1.0

_NC = 2
_NS = 16
_NW = _NC * _NS

_CHUNK = 512
_GRP = 16


def _sc_gather(ent_idx, rel_idx, ent_embs, rel_emb):
    n_ent_rows = ent_idx.shape[0]
    n_rel_rows = rel_idx.shape[0]
    ent_per_w = n_ent_rows // _NW
    rel_per_w = n_rel_rows // _NW
    mesh = plsc.VectorSubcoreMesh(core_axis_name="c", subcore_axis_name="s")

    @functools.partial(
        pl.kernel,
        out_type=(
            jax.ShapeDtypeStruct((n_ent_rows, _DIM), jnp.float32),
            jax.ShapeDtypeStruct((n_rel_rows, _DIM), jnp.float32),
        ),
        mesh=mesh,
        scratch_types=[
            pltpu.VMEM((_CHUNK,), jnp.int32),
            pltpu.VMEM((_CHUNK, _DIM), jnp.float32),
            pltpu.SemaphoreType.DMA,
        ],
    )
    def gather_kernel(ent_idx_hbm, rel_idx_hbm, ent_hbm, rel_hbm,
                      ent_out, rel_out, idx_v, rows_v, sem):
        wid = lax.axis_index("s") * _NC + lax.axis_index("c")

        def do_table(idx_hbm, tab_hbm, out_hbm, per_w):
            base = wid * per_w

            def chunk_body(c, carry):
                off = base + c * _CHUNK
                pltpu.sync_copy(idx_hbm.at[pl.ds(off, _CHUNK)], idx_v)

                def grp_fire(g, carry):
                    gbase = pl.multiple_of(g * _GRP, _GRP)
                    vec = idx_v[pl.ds(gbase, _GRP)]
                    for l in range(_GRP):
                        pltpu.make_async_copy(
                            tab_hbm.at[pl.ds(vec[l], 1), :],
                            rows_v.at[pl.ds(gbase + l, 1), :],
                            sem,
                        ).start()
                    return carry

                lax.fori_loop(0, _CHUNK // _GRP, grp_fire, jnp.int32(0))

                def grp_drain(g, carry):
                    pltpu.make_async_copy(
                        tab_hbm.at[pl.ds(0, 1), :],
                        rows_v.at[pl.ds(0, 1), :],
                        sem,
                    ).wait()
                    return carry

                lax.fori_loop(0, _CHUNK, grp_drain, jnp.int32(0))
                pltpu.sync_copy(rows_v, out_hbm.at[pl.ds(off, _CHUNK)])
                return carry

            lax.fori_loop(0, per_w // _CHUNK, chunk_body, jnp.int32(0))

        do_table(ent_idx_hbm, ent_hbm, ent_out, ent_per_w)
        do_table(rel_idx_hbm, rel_hbm, rel_out, rel_per_w)

    return gather_kernel(ent_idx, rel_idx, ent_embs, rel_emb)


_BLK = 2048


def _dense_body(hp_ref, tp_ref, hn_ref, tn_ref, r_ref, out_ref):
    i = pl.program_id(0)

    def nrm(x):
        n = jnp.sqrt(jnp.sum(x * x, axis=1, keepdims=True))
        return x / jnp.maximum(n, 1e-12)

    hp = nrm(hp_ref[...])
    tp = nrm(tp_ref[...])
    hn = nrm(hn_ref[...])
    tn = nrm(tn_ref[...])
    r = nrm(r_ref[...])
    pos_e = jnp.sqrt(jnp.sum((hp + r - tp) ** 2, axis=1))
    neg_e = jnp.sqrt(jnp.sum((hn + r - tn) ** 2, axis=1))
    loss = jnp.maximum(_MARGIN + pos_e - neg_e, 0.0)
    s = jnp.sum(loss)

    @pl.when(i == 0)
    def _init():
        out_ref[0, 0] = s

    @pl.when(i != 0)
    def _acc():
        out_ref[0, 0] += s

    @pl.when(i == pl.num_programs(0) - 1)
    def _final():
        out_ref[0, 0] = out_ref[0, 0] / _B


def _dense_loss(ent_rows, rel_rows):
    grid = _B // _BLK
    nblk = grid

    def section(k):
        return pl.BlockSpec((_BLK, _DIM), lambda i, k=k: (k * nblk + i, 0))

    return pl.pallas_call(
        _dense_body,
        grid=(grid,),
        in_specs=[section(0), section(1), section(2), section(3),
                  pl.BlockSpec((_BLK, _DIM), lambda i: (i, 0))],
        out_specs=pl.BlockSpec((1, 1), lambda i: (0, 0),
                               memory_space=pltpu.SMEM),
        out_shape=jax.ShapeDtypeStruct((1, 1), jnp.float32),
    )(ent_rows, ent_rows, ent_rows, ent_rows, rel_rows)


def kernel(pos_pairs, neg_pairs, rels, ent_embs, alignments, rel_emb):
    ent_idx = jnp.concatenate(
        [pos_pairs[:, 0], pos_pairs[:, 1], neg_pairs[:, 0], neg_pairs[:, 1]]
    )
    rel_idx = rels[:, 0]
    ent_rows, rel_rows = _sc_gather(ent_idx, rel_idx, ent_embs, rel_emb)
    out = _dense_loss(ent_rows, rel_rows)
    return out[0, 0]
